# baseline (device time: 91702 ns/iter reference)
import jax
import jax.numpy as jnp
from jax import lax
from jax.experimental import pallas as pl
from jax.experimental.pallas import tpu as pltpu

N_DEV = 16
PRECISION = lax.Precision.HIGHEST


def kernel(x, w_mat):
    k_dim, k_shard = x.shape
    n = w_mat.shape[1]
    m_blk = k_dim // N_DEV

    def body(x_ref, w_ref, out_ref, gather_ref, y_ref, amax_ref,
             send_sems, recv_sems, send_sems2, recv_sems2):
        me = lax.axis_index("i")

        barrier_sem = pltpu.get_barrier_semaphore()
        for d in range(N_DEV):
            @pl.when(me != d)
            def _():
                pl.semaphore_signal(
                    barrier_sem, inc=1,
                    device_id=(d,), device_id_type=pl.DeviceIdType.MESH,
                )
        pl.semaphore_wait(barrier_sem, N_DEV - 1)

        for r in range(1, N_DEV):
            t = (me + r) % N_DEV
            rdma = pltpu.make_async_remote_copy(
                src_ref=x_ref.at[pl.ds(t * m_blk, m_blk), :],
                dst_ref=gather_ref.at[:, pl.ds(me * k_shard, k_shard)],
                send_sem=send_sems.at[t],
                recv_sem=recv_sems.at[me],
                device_id=(t,),
                device_id_type=pl.DeviceIdType.MESH,
            )
            rdma.start()

        gather_ref[:, pl.ds(me * k_shard, k_shard)] = (
            x_ref[pl.ds(me * m_blk, m_blk), :])
        y_ref[:, :] = jnp.dot(
            gather_ref[:, pl.ds(me * k_shard, k_shard)],
            w_ref[pl.ds(me * k_shard, k_shard), :],
            preferred_element_type=jnp.float32, precision=PRECISION)

        for r in range(1, N_DEV):
            s = (me - r) % N_DEV
            rdma = pltpu.make_async_remote_copy(
                src_ref=x_ref.at[pl.ds(s * m_blk, m_blk), :],
                dst_ref=gather_ref.at[:, pl.ds(s * k_shard, k_shard)],
                send_sem=send_sems.at[s],
                recv_sem=recv_sems.at[s],
                device_id=(s,),
                device_id_type=pl.DeviceIdType.MESH,
            )
            rdma.wait_recv()
            y_ref[:, :] += jnp.dot(
                gather_ref[:, pl.ds(s * k_shard, k_shard)],
                w_ref[pl.ds(s * k_shard, k_shard), :],
                preferred_element_type=jnp.float32, precision=PRECISION)

        y = jnp.maximum(y_ref[:, :], 0.0)
        y_ref[:, :] = y

        for r in range(1, N_DEV):
            t = (me + r) % N_DEV
            rdma = pltpu.make_async_remote_copy(
                src_ref=x_ref.at[pl.ds(t * m_blk, m_blk), :],
                dst_ref=gather_ref.at[:, pl.ds(t * k_shard, k_shard)],
                send_sem=send_sems.at[t],
                recv_sem=recv_sems.at[t],
                device_id=(t,),
                device_id_type=pl.DeviceIdType.MESH,
            )
            rdma.wait_send()

        amax_ref[pl.ds(me, 1)] = jnp.full((1, 8, 128), jnp.max(y),
                                          dtype=jnp.float32)
        for r in range(1, N_DEV):
            t = (me + r) % N_DEV
            rdma = pltpu.make_async_remote_copy(
                src_ref=amax_ref.at[pl.ds(me, 1)],
                dst_ref=amax_ref.at[pl.ds(me, 1)],
                send_sem=send_sems2.at[t],
                recv_sem=recv_sems2.at[me],
                device_id=(t,),
                device_id_type=pl.DeviceIdType.MESH,
            )
            rdma.start()
        for r in range(1, N_DEV):
            s = (me - r) % N_DEV
            rdma = pltpu.make_async_remote_copy(
                src_ref=amax_ref.at[pl.ds(s, 1)],
                dst_ref=amax_ref.at[pl.ds(s, 1)],
                send_sem=send_sems2.at[s],
                recv_sem=recv_sems2.at[s],
                device_id=(s,),
                device_id_type=pl.DeviceIdType.MESH,
            )
            rdma.wait_recv()
            rdma.wait_send()

        gmax = jnp.max(amax_ref[:, :, :])
        scale = gmax / 448.0
        q = (y_ref[:, :] / scale).astype(jnp.float8_e4m3fn)
        out_ref[:, :] = q.astype(jnp.float32) * scale

    return pl.pallas_call(
        body,
        out_shape=jax.ShapeDtypeStruct((m_blk, n), jnp.float32),
        in_specs=[
            pl.BlockSpec(memory_space=pltpu.VMEM),
            pl.BlockSpec(memory_space=pltpu.VMEM),
        ],
        out_specs=pl.BlockSpec(memory_space=pltpu.VMEM),
        scratch_shapes=[
            pltpu.VMEM((m_blk, k_dim), jnp.float32),
            pltpu.VMEM((m_blk, n), jnp.float32),
            pltpu.VMEM((N_DEV, 8, 128), jnp.float32),
            pltpu.SemaphoreType.DMA((N_DEV,)),
            pltpu.SemaphoreType.DMA((N_DEV,)),
            pltpu.SemaphoreType.DMA((N_DEV,)),
            pltpu.SemaphoreType.DMA((N_DEV,)),
        ],
        compiler_params=pltpu.CompilerParams(
            collective_id=0,
            vmem_limit_bytes=100 * 1024 * 1024,
        ),
    )(x, w_mat)


# device time: 56692 ns/iter; 1.6175x vs baseline; 1.6175x over previous
import jax
import jax.numpy as jnp
from jax import lax
from jax.experimental import pallas as pl
from jax.experimental.pallas import tpu as pltpu

N_DEV = 16
PRECISION = None


def kernel(x, w_mat):
    k_dim, k_shard = x.shape
    n = w_mat.shape[1]
    m_blk = k_dim // N_DEV

    def body(x_ref, w_ref, out_ref, x16_ref, gather_ref, y_ref, amax_ref,
             send_sems, recv_sems, send_sems2, recv_sems2):
        me = lax.axis_index("i")

        x16_ref[:, :] = x_ref[:, :].astype(jnp.bfloat16)

        barrier_sem = pltpu.get_barrier_semaphore()
        for d in range(N_DEV):
            @pl.when(me != d)
            def _():
                pl.semaphore_signal(
                    barrier_sem, inc=1,
                    device_id=(d,), device_id_type=pl.DeviceIdType.MESH,
                )
        pl.semaphore_wait(barrier_sem, N_DEV - 1)

        for r in range(1, N_DEV):
            t = (me + r) % N_DEV
            rdma = pltpu.make_async_remote_copy(
                src_ref=x16_ref.at[pl.ds(t * m_blk, m_blk), :],
                dst_ref=gather_ref.at[:, pl.ds(me * k_shard, k_shard)],
                send_sem=send_sems.at[t],
                recv_sem=recv_sems.at[me],
                device_id=(t,),
                device_id_type=pl.DeviceIdType.MESH,
            )
            rdma.start()

        gather_ref[:, pl.ds(me * k_shard, k_shard)] = (
            x16_ref[pl.ds(me * m_blk, m_blk), :])
        y_ref[:, :] = jnp.dot(
            gather_ref[:, pl.ds(me * k_shard, k_shard)],
            w_ref[pl.ds(me * k_shard, k_shard), :],
            preferred_element_type=jnp.float32, precision=PRECISION)

        for r in range(1, N_DEV):
            s = (me - r) % N_DEV
            rdma = pltpu.make_async_remote_copy(
                src_ref=x16_ref.at[pl.ds(s * m_blk, m_blk), :],
                dst_ref=gather_ref.at[:, pl.ds(s * k_shard, k_shard)],
                send_sem=send_sems.at[s],
                recv_sem=recv_sems.at[s],
                device_id=(s,),
                device_id_type=pl.DeviceIdType.MESH,
            )
            rdma.wait_recv()
            y_ref[:, :] += jnp.dot(
                gather_ref[:, pl.ds(s * k_shard, k_shard)],
                w_ref[pl.ds(s * k_shard, k_shard), :],
                preferred_element_type=jnp.float32, precision=PRECISION)

        y = jnp.maximum(y_ref[:, :], 0.0)
        y_ref[:, :] = y

        for r in range(1, N_DEV):
            t = (me + r) % N_DEV
            rdma = pltpu.make_async_remote_copy(
                src_ref=x16_ref.at[pl.ds(t * m_blk, m_blk), :],
                dst_ref=gather_ref.at[:, pl.ds(t * k_shard, k_shard)],
                send_sem=send_sems.at[t],
                recv_sem=recv_sems.at[t],
                device_id=(t,),
                device_id_type=pl.DeviceIdType.MESH,
            )
            rdma.wait_send()

        amax_ref[pl.ds(me, 1)] = jnp.full((1, 8, 128), jnp.max(y),
                                          dtype=jnp.float32)
        for r in range(1, N_DEV):
            t = (me + r) % N_DEV
            rdma = pltpu.make_async_remote_copy(
                src_ref=amax_ref.at[pl.ds(me, 1)],
                dst_ref=amax_ref.at[pl.ds(me, 1)],
                send_sem=send_sems2.at[t],
                recv_sem=recv_sems2.at[me],
                device_id=(t,),
                device_id_type=pl.DeviceIdType.MESH,
            )
            rdma.start()
        for r in range(1, N_DEV):
            s = (me - r) % N_DEV
            rdma = pltpu.make_async_remote_copy(
                src_ref=amax_ref.at[pl.ds(s, 1)],
                dst_ref=amax_ref.at[pl.ds(s, 1)],
                send_sem=send_sems2.at[s],
                recv_sem=recv_sems2.at[s],
                device_id=(s,),
                device_id_type=pl.DeviceIdType.MESH,
            )
            rdma.wait_recv()
            rdma.wait_send()

        gmax = jnp.max(amax_ref[:, :, :])
        scale = gmax / 448.0
        q = (y_ref[:, :] / scale).astype(jnp.float8_e4m3fn)
        out_ref[:, :] = q.astype(jnp.float32) * scale

    return pl.pallas_call(
        body,
        out_shape=jax.ShapeDtypeStruct((m_blk, n), jnp.float32),
        in_specs=[
            pl.BlockSpec(memory_space=pltpu.VMEM),
            pl.BlockSpec(memory_space=pltpu.VMEM),
        ],
        out_specs=pl.BlockSpec(memory_space=pltpu.VMEM),
        scratch_shapes=[
            pltpu.VMEM((k_dim, k_shard), jnp.bfloat16),
            pltpu.VMEM((m_blk, k_dim), jnp.bfloat16),
            pltpu.VMEM((m_blk, n), jnp.float32),
            pltpu.VMEM((N_DEV, 8, 128), jnp.float32),
            pltpu.SemaphoreType.DMA((N_DEV,)),
            pltpu.SemaphoreType.DMA((N_DEV,)),
            pltpu.SemaphoreType.DMA((N_DEV,)),
            pltpu.SemaphoreType.DMA((N_DEV,)),
        ],
        compiler_params=pltpu.CompilerParams(
            collective_id=0,
            vmem_limit_bytes=100 * 1024 * 1024,
        ),
    )(x, w_mat)


# device time: 50128 ns/iter; 1.8294x vs baseline; 1.1309x over previous
import jax
import jax.numpy as jnp
from jax import lax
from jax.experimental import pallas as pl
from jax.experimental.pallas import tpu as pltpu

N_DEV = 16


def kernel(x, w_mat):
    k_dim, k_shard = x.shape
    n = w_mat.shape[1]
    m_blk = k_dim // N_DEV

    def body(x_ref, w_ref, out_ref, x16_ref, gather_ref, y_ref, w_buf,
             amax_ref, send_sems, recv_sems, send_sems2, recv_sems2, w_sems):
        me = lax.axis_index("i")

        def w_block_copy(src_blk, slot):
            return pltpu.make_async_copy(
                w_ref.at[pl.ds(src_blk * k_shard, k_shard), :],
                w_buf.at[slot], w_sems.at[slot])

        w_block_copy(me, 0).start()

        x16_ref[:, :] = x_ref[:, :].astype(jnp.bfloat16)

        barrier_sem = pltpu.get_barrier_semaphore()
        for d in range(N_DEV):
            @pl.when(me != d)
            def _():
                pl.semaphore_signal(
                    barrier_sem, inc=1,
                    device_id=(d,), device_id_type=pl.DeviceIdType.MESH,
                )
        pl.semaphore_wait(barrier_sem, N_DEV - 1)

        for r in range(1, N_DEV):
            t = (me + r) % N_DEV
            rdma = pltpu.make_async_remote_copy(
                src_ref=x16_ref.at[pl.ds(t * m_blk, m_blk), :],
                dst_ref=gather_ref.at[:, pl.ds(me * k_shard, k_shard)],
                send_sem=send_sems.at[t],
                recv_sem=recv_sems.at[me],
                device_id=(t,),
                device_id_type=pl.DeviceIdType.MESH,
            )
            rdma.start()

        gather_ref[:, pl.ds(me * k_shard, k_shard)] = (
            x16_ref[pl.ds(me * m_blk, m_blk), :])

        w_block_copy((me - 1) % N_DEV, 1).start()
        w_block_copy(me, 0).wait()
        y_ref[:, :] = jnp.dot(
            gather_ref[:, pl.ds(me * k_shard, k_shard)], w_buf[0],
            preferred_element_type=jnp.float32)

        for r in range(1, N_DEV):
            s = (me - r) % N_DEV
            slot = r % 2
            if r < N_DEV - 1:
                w_block_copy((me - r - 1) % N_DEV, (r + 1) % 2).start()
            w_block_copy(s, slot).wait()
            rdma = pltpu.make_async_remote_copy(
                src_ref=x16_ref.at[pl.ds(s * m_blk, m_blk), :],
                dst_ref=gather_ref.at[:, pl.ds(s * k_shard, k_shard)],
                send_sem=send_sems.at[s],
                recv_sem=recv_sems.at[s],
                device_id=(s,),
                device_id_type=pl.DeviceIdType.MESH,
            )
            rdma.wait_recv()
            y_ref[:, :] += jnp.dot(
                gather_ref[:, pl.ds(s * k_shard, k_shard)], w_buf[slot],
                preferred_element_type=jnp.float32)

        y = jnp.maximum(y_ref[:, :], 0.0)
        y_ref[:, :] = y

        amax_ref[pl.ds(me, 1)] = jnp.full((1, 8, 128), jnp.max(y),
                                          dtype=jnp.float32)
        for r in range(1, N_DEV):
            t = (me + r) % N_DEV
            rdma = pltpu.make_async_remote_copy(
                src_ref=amax_ref.at[pl.ds(me, 1)],
                dst_ref=amax_ref.at[pl.ds(me, 1)],
                send_sem=send_sems2.at[t],
                recv_sem=recv_sems2.at[me],
                device_id=(t,),
                device_id_type=pl.DeviceIdType.MESH,
            )
            rdma.start()

        for r in range(1, N_DEV):
            t = (me + r) % N_DEV
            rdma = pltpu.make_async_remote_copy(
                src_ref=x16_ref.at[pl.ds(t * m_blk, m_blk), :],
                dst_ref=gather_ref.at[:, pl.ds(t * k_shard, k_shard)],
                send_sem=send_sems.at[t],
                recv_sem=recv_sems.at[t],
                device_id=(t,),
                device_id_type=pl.DeviceIdType.MESH,
            )
            rdma.wait_send()

        for r in range(1, N_DEV):
            s = (me - r) % N_DEV
            rdma = pltpu.make_async_remote_copy(
                src_ref=amax_ref.at[pl.ds(s, 1)],
                dst_ref=amax_ref.at[pl.ds(s, 1)],
                send_sem=send_sems2.at[s],
                recv_sem=recv_sems2.at[s],
                device_id=(s,),
                device_id_type=pl.DeviceIdType.MESH,
            )
            rdma.wait_recv()
            rdma.wait_send()

        gmax = jnp.max(amax_ref[:, :, :])
        scale = gmax / 448.0
        q = (y_ref[:, :] / scale).astype(jnp.float8_e4m3fn)
        out_ref[:, :] = q.astype(jnp.float32) * scale

    return pl.pallas_call(
        body,
        out_shape=jax.ShapeDtypeStruct((m_blk, n), jnp.float32),
        in_specs=[
            pl.BlockSpec(memory_space=pltpu.VMEM),
            pl.BlockSpec(memory_space=pl.ANY),
        ],
        out_specs=pl.BlockSpec(memory_space=pltpu.VMEM),
        scratch_shapes=[
            pltpu.VMEM((k_dim, k_shard), jnp.bfloat16),
            pltpu.VMEM((m_blk, k_dim), jnp.bfloat16),
            pltpu.VMEM((m_blk, n), jnp.float32),
            pltpu.VMEM((2, k_shard, n), jnp.float32),
            pltpu.VMEM((N_DEV, 8, 128), jnp.float32),
            pltpu.SemaphoreType.DMA((N_DEV,)),
            pltpu.SemaphoreType.DMA((N_DEV,)),
            pltpu.SemaphoreType.DMA((N_DEV,)),
            pltpu.SemaphoreType.DMA((N_DEV,)),
            pltpu.SemaphoreType.DMA((2,)),
        ],
        compiler_params=pltpu.CompilerParams(
            collective_id=0,
            vmem_limit_bytes=100 * 1024 * 1024,
        ),
    )(x, w_mat)
